# baseline (speedup 1.0000x reference)
import functools

import jax
import jax.numpy as jnp
from jax import lax
from jax.experimental import pallas as pl
from jax.experimental.pallas import tpu as pltpu
from jax.experimental.pallas import tpu_sc as plsc

VOCAB = 1000000
DIM = 64
B = 16384
L = 50

NC = 2
NS = 16
NW = NC * NS

CHUNK = 128
DT = DIM // 8
BT = B // CHUNK
UNITS = L * BT
UPW = UNITS // NW
NBUF = 8
NR = UPW // NBUF


def _gather_kernel(idx_hbm, table_hbm, out_hbm, idx_v, *rest):
    gbufs = rest[0:NBUF]
    tbufs = rest[NBUF:NBUF + 2]
    gsem = rest[NBUF + 2:2 * NBUF + 2]
    osem = rest[2 * NBUF + 2:2 * NBUF + 4]
    wid = lax.axis_index("c") * NS + lax.axis_index("s")
    ubase = wid * UPW
    pltpu.sync_copy(idx_hbm.at[pl.ds(ubase, UPW)], idx_v)

    lane = lax.iota(jnp.int32, 16)
    ridx = [lane + jb * 16 for jb in range(8)]

    for j in range(NBUF):
        pltpu.async_copy(table_hbm.at[idx_v.at[j]], gbufs[j], gsem[j])

    def do_unit(r, jj):
        j = r * NBUF + jj
        u = ubase + j
        l = u // BT
        bt = u % BT
        slot = jj
        par = jj % 2
        gbuf = gbufs[slot]
        tbuf = tbufs[par]

        pltpu.make_async_copy(
            table_hbm.at[idx_v.at[j]], gbuf, gsem[slot]).wait()

        def drain_outs(j=j):
            up = ubase + j - 2
            lp = up // BT
            btp = up % BT
            for dt in range(DT):
                pltpu.make_async_copy(
                    tbufs[par].at[pl.ds(dt * 8, 8)],
                    out_hbm.at[lp, dt, btp], osem[par]).wait()

        if jj >= 2:
            drain_outs()
        else:
            @pl.when(r > 0)
            def _():
                drain_outs()

        def trans_body(d, carry):
            cidx = jnp.zeros((16,), jnp.int32) + d
            for jb in range(8):
                tbuf[d, pl.ds(jb * 16, 16)] = plsc.load_gather(
                    gbuf, [ridx[jb], cidx])
            return carry

        lax.fori_loop(0, DIM, trans_body, 0)

        for dt in range(DT):
            pltpu.async_copy(
                tbuf.at[pl.ds(dt * 8, 8)], out_hbm.at[l, dt, bt], osem[par])

        @pl.when(r < NR - 1)
        def _():
            pltpu.async_copy(
                table_hbm.at[idx_v.at[j + NBUF]], gbufs[slot], gsem[slot])

    def body(r, carry):
        for jj in range(NBUF):
            do_unit(r, jj)
        return carry

    lax.fori_loop(0, NR, body, 0)

    for j in (UPW - 2, UPW - 1):
        u = ubase + j
        l = u // BT
        bt = u % BT
        for dt in range(DT):
            pltpu.make_async_copy(
                tbufs[j % 2].at[pl.ds(dt * 8, 8)],
                out_hbm.at[l, dt, bt], osem[j % 2]).wait()


@jax.jit
def _embedding_lookup(xt_chunks, embedding):
    mesh = plsc.VectorSubcoreMesh(core_axis_name="c", subcore_axis_name="s")
    fn = functools.partial(
        pl.kernel,
        mesh=mesh,
        out_type=jax.ShapeDtypeStruct((L, DT, BT, 8, CHUNK), jnp.float32),
        scratch_types=(
            [pltpu.VMEM((UPW, CHUNK), jnp.int32)]
            + [pltpu.VMEM((CHUNK, DIM), jnp.float32) for _ in range(NBUF)]
            + [pltpu.VMEM((DIM, CHUNK), jnp.float32) for _ in range(2)]
            + [pltpu.SemaphoreType.DMA for _ in range(NBUF + 2)]
        ),
        compiler_params=pltpu.CompilerParams(
            use_tc_tiling_on_sc=False, needs_layout_passes=False),
    )(_gather_kernel)
    return fn(xt_chunks, embedding)


def kernel(x, embedding):
    xt_chunks = x.T.reshape(UNITS, CHUNK)
    out5 = _embedding_lookup(xt_chunks, embedding)
    return out5.transpose(2, 4, 0, 1, 3).reshape(B, L, DIM)


# speedup vs baseline: 1.4738x; 1.4738x over previous
import functools

import jax
import jax.numpy as jnp
from jax import lax
from jax.experimental import pallas as pl
from jax.experimental.pallas import tpu as pltpu
from jax.experimental.pallas import tpu_sc as plsc

VOCAB = 1000000
DIM = 64
B = 16384
L = 50

NC = 2
NS = 16
NW = NC * NS

CHUNK = 128
DT = DIM // 8
BT = B // CHUNK
UNITS = L * BT
UPW = UNITS // NW
NBUF = 8
NR = UPW // NBUF


def _gather_kernel(idx_hbm, table_hbm, out_hbm, idx_v, *rest):
    gbufs = rest[0:NBUF]
    tbufs = rest[NBUF:NBUF + 2]
    gsem = rest[NBUF + 2:2 * NBUF + 2]
    osem = rest[2 * NBUF + 2:2 * NBUF + 4]
    wid = lax.axis_index("c") * NS + lax.axis_index("s")
    ubase = wid * UPW
    pltpu.sync_copy(idx_hbm.at[pl.ds(ubase, UPW)], idx_v)

    lane = lax.iota(jnp.int32, 16)
    perm = [(lane + k) % 16 for k in range(16)]

    for j in range(NBUF):
        pltpu.async_copy(table_hbm.at[idx_v.at[j]], gbufs[j], gsem[j])

    def do_unit(r, jj):
        j = r * NBUF + jj
        u = ubase + j
        l = u // BT
        bt = u % BT
        slot = jj
        par = jj % 2
        gbuf = gbufs[slot]
        tbuf = tbufs[par]

        pltpu.make_async_copy(
            table_hbm.at[idx_v.at[j]], gbuf, gsem[slot]).wait()

        def drain_outs(j=j):
            up = ubase + j - 2
            lp = up // BT
            btp = up % BT
            for dt in range(DT):
                pltpu.make_async_copy(
                    tbufs[par].at[pl.ds(dt * 8, 8)],
                    out_hbm.at[lp, dt, btp], osem[par]).wait()

        if jj >= 2:
            drain_outs()
        else:
            @pl.when(r > 0)
            def _():
                drain_outs()

        def trans_body(r0, carry):
            rr = r0 * 16 + lane
            for d0 in range(DIM // 16):
                for k in range(16):
                    dd = perm[k] + d0 * 16
                    plsc.store_scatter(
                        tbuf, [dd, rr], plsc.load_gather(gbuf, [rr, dd]))
            return carry

        lax.fori_loop(0, CHUNK // 16, trans_body, 0)

        for dt in range(DT):
            pltpu.async_copy(
                tbuf.at[pl.ds(dt * 8, 8)], out_hbm.at[l, dt, bt], osem[par])

        @pl.when(r < NR - 1)
        def _():
            pltpu.async_copy(
                table_hbm.at[idx_v.at[j + NBUF]], gbufs[slot], gsem[slot])

    def body(r, carry):
        for jj in range(NBUF):
            do_unit(r, jj)
        return carry

    lax.fori_loop(0, NR, body, 0)

    for j in (UPW - 2, UPW - 1):
        u = ubase + j
        l = u // BT
        bt = u % BT
        for dt in range(DT):
            pltpu.make_async_copy(
                tbufs[j % 2].at[pl.ds(dt * 8, 8)],
                out_hbm.at[l, dt, bt], osem[j % 2]).wait()


@jax.jit
def _embedding_lookup(xt_chunks, embedding):
    mesh = plsc.VectorSubcoreMesh(core_axis_name="c", subcore_axis_name="s")
    fn = functools.partial(
        pl.kernel,
        mesh=mesh,
        out_type=jax.ShapeDtypeStruct((L, DT, BT, 8, CHUNK), jnp.float32),
        scratch_types=(
            [pltpu.VMEM((UPW, CHUNK), jnp.int32)]
            + [pltpu.VMEM((CHUNK, DIM), jnp.float32) for _ in range(NBUF)]
            + [pltpu.VMEM((DIM, CHUNK), jnp.float32) for _ in range(2)]
            + [pltpu.SemaphoreType.DMA for _ in range(NBUF + 2)]
        ),
        compiler_params=pltpu.CompilerParams(
            use_tc_tiling_on_sc=False, needs_layout_passes=False),
    )(_gather_kernel)
    return fn(xt_chunks, embedding)


def kernel(x, embedding):
    xt_chunks = x.T.reshape(UNITS, CHUNK)
    out5 = _embedding_lookup(xt_chunks, embedding)
    return out5.transpose(2, 4, 0, 1, 3).reshape(B, L, DIM)


# speedup vs baseline: 1.8175x; 1.2332x over previous
import functools

import jax
import jax.numpy as jnp
from jax import lax
from jax.experimental import pallas as pl
from jax.experimental.pallas import tpu as pltpu
from jax.experimental.pallas import tpu_sc as plsc

VOCAB = 1000000
DIM = 64
B = 16384
L = 50

NC = 2
NS = 16
NW = NC * NS

CHUNK = 128
DT = DIM // 8
BT = B // CHUNK
UNITS = L * BT
UPW = UNITS // NW
NBUF = 4
NR = UPW // NBUF


def _gather_kernel(idx_hbm, table_hbm, out_hbm, idx_v, *rest):
    gbufs = rest[0:NBUF]
    tbufs = rest[NBUF:NBUF + 2]
    gsem = rest[NBUF + 2:2 * NBUF + 2]
    osem = rest[2 * NBUF + 2:2 * NBUF + 4]
    wid = lax.axis_index("c") * NS + lax.axis_index("s")
    ubase = wid * UPW
    pltpu.sync_copy(idx_hbm.at[pl.ds(ubase, UPW)], idx_v)

    lane = lax.iota(jnp.int32, 16)
    perm = [(lane + k) % 16 for k in range(16)]
    rr = [lane + r0 * 16 for r0 in range(CHUNK // 16)]

    for j in range(NBUF):
        pltpu.async_copy(table_hbm.at[idx_v.at[j]], gbufs[j], gsem[j])

    def do_unit(r, jj):
        j = r * NBUF + jj
        u = ubase + j
        l = u // BT
        bt = u % BT
        slot = jj
        par = jj % 2
        gbuf = gbufs[slot]
        tbuf = tbufs[par]

        pltpu.make_async_copy(
            table_hbm.at[idx_v.at[j]], gbuf, gsem[slot]).wait()

        def drain_outs(j=j):
            up = ubase + j - 2
            lp = up // BT
            btp = up % BT
            for dt in range(DT):
                pltpu.make_async_copy(
                    tbufs[par].at[pl.ds(dt * 8, 8)],
                    out_hbm.at[lp, dt, btp], osem[par]).wait()

        if jj >= 2:
            drain_outs()
        else:
            @pl.when(r > 0)
            def _():
                drain_outs()

        def trans_body(d0, carry):
            off = d0 * 16
            for k in range(16):
                dd = perm[k] + off
                for r0 in range(CHUNK // 16):
                    plsc.store_scatter(
                        tbuf, [dd, rr[r0]],
                        plsc.load_gather(gbuf, [rr[r0], dd]))
            return carry

        lax.fori_loop(0, DIM // 16, trans_body, 0)

        for dt in range(DT):
            pltpu.async_copy(
                tbuf.at[pl.ds(dt * 8, 8)], out_hbm.at[l, dt, bt], osem[par])

        @pl.when(r < NR - 1)
        def _():
            pltpu.async_copy(
                table_hbm.at[idx_v.at[j + NBUF]], gbufs[slot], gsem[slot])

    def body(r, carry):
        for jj in range(NBUF):
            do_unit(r, jj)
        return carry

    lax.fori_loop(0, NR, body, 0)

    for j in (UPW - 2, UPW - 1):
        u = ubase + j
        l = u // BT
        bt = u % BT
        for dt in range(DT):
            pltpu.make_async_copy(
                tbufs[j % 2].at[pl.ds(dt * 8, 8)],
                out_hbm.at[l, dt, bt], osem[j % 2]).wait()


@jax.jit
def _embedding_lookup(xt_chunks, embedding):
    mesh = plsc.VectorSubcoreMesh(core_axis_name="c", subcore_axis_name="s")
    fn = functools.partial(
        pl.kernel,
        mesh=mesh,
        out_type=jax.ShapeDtypeStruct((L, DT, BT, 8, CHUNK), jnp.float32),
        scratch_types=(
            [pltpu.VMEM((UPW, CHUNK), jnp.int32)]
            + [pltpu.VMEM((CHUNK, DIM), jnp.float32) for _ in range(NBUF)]
            + [pltpu.VMEM((DIM, CHUNK), jnp.float32) for _ in range(2)]
            + [pltpu.SemaphoreType.DMA for _ in range(NBUF + 2)]
        ),
        compiler_params=pltpu.CompilerParams(
            use_tc_tiling_on_sc=False, needs_layout_passes=False),
    )(_gather_kernel)
    return fn(xt_chunks, embedding)


def kernel(x, embedding):
    xt_chunks = x.T.reshape(UNITS, CHUNK)
    out5 = _embedding_lookup(xt_chunks, embedding)
    return out5.transpose(2, 4, 0, 1, 3).reshape(B, L, DIM)


# speedup vs baseline: 2.4276x; 1.3357x over previous
import functools

import jax
import jax.numpy as jnp
from jax import lax
from jax.experimental import pallas as pl
from jax.experimental.pallas import tpu as pltpu
from jax.experimental.pallas import tpu_sc as plsc

VOCAB = 1000000
DIM = 64
B = 16384
L = 50

NC = 2
NS = 16
NW = NC * NS

CHUNK = 128
DT = DIM // 8
BT = B // CHUNK
UNITS = L * BT
UPW = UNITS // NW
NBUF = 4
NR = UPW // NBUF


def _gather_kernel(idx_hbm, table_hbm, out_hbm, idx_v, *rest):
    gbufs = rest[0:NBUF]
    tbufs = rest[NBUF:NBUF + 2]
    gsem = rest[NBUF + 2:2 * NBUF + 2]
    osem = rest[2 * NBUF + 2:2 * NBUF + 4]
    wid = lax.axis_index("c") * NS + lax.axis_index("s")
    ubase = wid * UPW
    pltpu.sync_copy(idx_hbm.at[pl.ds(ubase, UPW)], idx_v)

    lane = lax.iota(jnp.int32, 16)
    rr = [lane + r0 * 16 for r0 in range(CHUNK // 16)]

    for j in range(NBUF):
        pltpu.async_copy(table_hbm.at[idx_v.at[j]], gbufs[j], gsem[j])

    def do_unit(r, jj):
        j = r * NBUF + jj
        u = ubase + j
        l = u // BT
        bt = u % BT
        slot = jj
        par = jj % 2
        gbuf = gbufs[slot]
        tbuf = tbufs[par]

        pltpu.make_async_copy(
            table_hbm.at[idx_v.at[j]], gbuf, gsem[slot]).wait()

        def drain_outs(j=j):
            up = ubase + j - 2
            lp = up // BT
            btp = up % BT
            for dt in range(DT):
                pltpu.make_async_copy(
                    tbufs[par].at[pl.ds(dt * 8, 8)],
                    out_hbm.at[lp, dt, btp], osem[par]).wait()

        if jj >= 2:
            drain_outs()
        else:
            @pl.when(r > 0)
            def _():
                drain_outs()

        def trans_body(k, carry):
            dd0 = (lane + k) & 15
            for d0 in range(DIM // 16):
                dd = dd0 + d0 * 16
                vals = [plsc.load_gather(gbuf, [rr[r0], dd])
                        for r0 in range(CHUNK // 16)]
                for r0 in range(CHUNK // 16):
                    plsc.store_scatter(tbuf, [dd, rr[r0]], vals[r0])
            return carry

        lax.fori_loop(0, 16, trans_body, 0)

        for dt in range(DT):
            pltpu.async_copy(
                tbuf.at[pl.ds(dt * 8, 8)], out_hbm.at[l, dt, bt], osem[par])

        @pl.when(r < NR - 1)
        def _():
            pltpu.async_copy(
                table_hbm.at[idx_v.at[j + NBUF]], gbufs[slot], gsem[slot])

    def body(r, carry):
        for jj in range(NBUF):
            do_unit(r, jj)
        return carry

    lax.fori_loop(0, NR, body, 0)

    for j in (UPW - 2, UPW - 1):
        u = ubase + j
        l = u // BT
        bt = u % BT
        for dt in range(DT):
            pltpu.make_async_copy(
                tbufs[j % 2].at[pl.ds(dt * 8, 8)],
                out_hbm.at[l, dt, bt], osem[j % 2]).wait()


@jax.jit
def _embedding_lookup(xt_chunks, embedding):
    mesh = plsc.VectorSubcoreMesh(core_axis_name="c", subcore_axis_name="s")
    fn = functools.partial(
        pl.kernel,
        mesh=mesh,
        out_type=jax.ShapeDtypeStruct((L, DT, BT, 8, CHUNK), jnp.float32),
        scratch_types=(
            [pltpu.VMEM((UPW, CHUNK), jnp.int32)]
            + [pltpu.VMEM((CHUNK, DIM), jnp.float32) for _ in range(NBUF)]
            + [pltpu.VMEM((DIM, CHUNK), jnp.float32) for _ in range(2)]
            + [pltpu.SemaphoreType.DMA for _ in range(NBUF + 2)]
        ),
        compiler_params=pltpu.CompilerParams(
            use_tc_tiling_on_sc=False, needs_layout_passes=False),
    )(_gather_kernel)
    return fn(xt_chunks, embedding)


def kernel(x, embedding):
    xt_chunks = x.T.reshape(UNITS, CHUNK)
    out5 = _embedding_lookup(xt_chunks, embedding)
    return out5.transpose(2, 4, 0, 1, 3).reshape(B, L, DIM)
